# Initial kernel scaffold; baseline (speedup 1.0000x reference)
import jax
import jax.numpy as jnp
from jax.experimental import pallas as pl


def kernel(x):
    raise NotImplementedError("write your pallas kernel here")


# speedup vs baseline: 35.2455x; 35.2455x over previous
import functools

import jax
import jax.numpy as jnp
from jax import lax
from jax.experimental import pallas as pl
from jax.experimental.pallas import tpu as pltpu
from jax.experimental.pallas import tpu_sc as plsc

B, L, V = 16384, 200, 40
NC, NS, LANES = 2, 16, 16
NW = NC * NS
ROWS_PER_W = B // NW
GROUPS = ROWS_PER_W // LANES
POS_UNROLL = 8

_BIG = 1e30


def _body(x_hbm, out_hbm, xv, hist, outv):
    wid = lax.axis_index("s") * NC + lax.axis_index("c")
    row0 = wid * ROWS_PER_W
    pltpu.sync_copy(x_hbm.at[pl.ds(row0, ROWS_PER_W)], xv)

    lane = lax.broadcasted_iota(jnp.int32, (LANES,), 0)
    ones = jnp.ones((LANES,), jnp.float32)
    zf = jnp.zeros((LANES,), jnp.float32)

    for v in range(V):
        hist[v, :] = zf

    def group_body(g, carry):
        rbase = g * LANES
        ridx = rbase + lane

        def pos_body(i, c):
            for sub in range(POS_UNROLL):
                p = i * POS_UNROLL + sub
                cidx = jnp.full((LANES,), p, jnp.int32)
                ch = plsc.load_gather(xv, [ridx, cidx])
                plsc.addupdate_scatter(hist, [ch, lane], ones)
            return c

        lax.fori_loop(0, L // POS_UNROLL, pos_body, 0)

        lt = zf
        dg = zf
        sp = zf
        uq = zf
        mx = zf
        mn = jnp.full((LANES,), _BIG, jnp.float32)
        for v in range(1, V):
            h = hist[v, :]
            hist[v, :] = zf
            if v < 27:
                lt = lt + h
            elif v < 37:
                dg = dg + h
            else:
                sp = sp + h
            pos = h > 0.0
            uq = uq + jnp.where(pos, 1.0, 0.0)
            mx = jnp.maximum(mx, h)
            mn = jnp.minimum(mn, jnp.where(pos, h, _BIG))
        hist[0, :] = zf

        total = lt + dg + sp
        has = total > 0.0
        rec = 1.0 / jnp.where(has, total, 1.0)
        feats = (
            uq * jnp.float32(1.0 / V),
            mx * rec,
            jnp.where(has, mn, 0.0) * rec,
            lt * rec,
            dg * rec,
            sp * rec,
        )
        for fi, val in enumerate(feats):
            plsc.store_scatter(
                outv, [ridx, jnp.full((LANES,), fi, jnp.int32)], val)
        return carry

    lax.fori_loop(0, GROUPS, group_body, 0)
    pltpu.sync_copy(outv, out_hbm.at[pl.ds(row0, ROWS_PER_W)])


@jax.jit
def _analyze(x):
    mesh = plsc.VectorSubcoreMesh(core_axis_name="c", subcore_axis_name="s")
    return pl.kernel(
        _body,
        out_type=jax.ShapeDtypeStruct((B, 6), jnp.float32),
        mesh=mesh,
        scratch_types=[
            pltpu.VMEM((ROWS_PER_W, L), jnp.int32),
            pltpu.VMEM((V, LANES), jnp.float32),
            pltpu.VMEM((ROWS_PER_W, 6), jnp.float32),
        ],
        compiler_params=pltpu.CompilerParams(
            use_tc_tiling_on_sc=False, needs_layout_passes=False),
    )(x)


def kernel(x):
    return _analyze(x)


# speedup vs baseline: 43.1963x; 1.2256x over previous
import functools

import jax
import jax.numpy as jnp
from jax import lax
from jax.experimental import pallas as pl
from jax.experimental.pallas import tpu as pltpu
from jax.experimental.pallas import tpu_sc as plsc

B, L, V = 16384, 200, 40
NC, NS, LANES = 2, 16, 16
NW = NC * NS
ROWS_PER_W = B // NW
GROUPS = ROWS_PER_W // LANES
HIST_STRIDE = 64
POS_UNROLL = 8

_BIG = 1e30


def _body(x_hbm, out_hbm, xv, hist, outv):
    wid = lax.axis_index("s") * NC + lax.axis_index("c")
    row0 = wid * ROWS_PER_W
    pltpu.sync_copy(x_hbm.at[pl.ds(row0 * L, ROWS_PER_W * L)], xv)

    lane = lax.broadcasted_iota(jnp.int32, (LANES,), 0)
    lane_h = lane * HIST_STRIDE
    lane_x = lane * L
    lane_o = lane * 6
    ones = jnp.ones((LANES,), jnp.float32)
    zf = jnp.zeros((LANES,), jnp.float32)

    for k in range(LANES * HIST_STRIDE // LANES):
        hist[pl.ds(k * LANES, LANES)] = zf

    def group_body(g, carry):
        base = lane_x + g * (LANES * L)

        @plsc.parallel_loop(0, L, unroll=POS_UNROLL)
        def _pos(p):
            ch = plsc.load_gather(xv, [base + p])
            plsc.addupdate_scatter(hist, [lane_h + ch], ones)

        lt = zf
        dg = zf
        sp = zf
        uq = zf
        mx = zf
        mn = jnp.full((LANES,), _BIG, jnp.float32)
        for v in range(1, V):
            iv = lane_h + v
            h = plsc.load_gather(hist, [iv])
            plsc.store_scatter(hist, [iv], zf)
            if v < 27:
                lt = lt + h
            elif v < 37:
                dg = dg + h
            else:
                sp = sp + h
            uq = uq + jnp.minimum(h, ones)
            mx = jnp.maximum(mx, h)
            mn = jnp.minimum(mn, jnp.where(h > 0.0, h, _BIG))
        plsc.store_scatter(hist, [lane_h], zf)

        total = lt + dg + sp
        has = total > 0.0
        rec = 1.0 / jnp.where(has, total, 1.0)
        feats = (
            uq * jnp.float32(1.0 / V),
            mx * rec,
            jnp.where(has, mn, 0.0) * rec,
            lt * rec,
            dg * rec,
            sp * rec,
        )
        obase = lane_o + g * (LANES * 6)
        for fi, val in enumerate(feats):
            plsc.store_scatter(outv, [obase + fi], val)
        return carry

    lax.fori_loop(0, GROUPS, group_body, 0)
    pltpu.sync_copy(outv, out_hbm.at[pl.ds(row0 * 6, ROWS_PER_W * 6)])


@jax.jit
def _analyze(x):
    mesh = plsc.VectorSubcoreMesh(core_axis_name="c", subcore_axis_name="s")
    out = pl.kernel(
        _body,
        out_type=jax.ShapeDtypeStruct((B * 6,), jnp.float32),
        mesh=mesh,
        scratch_types=[
            pltpu.VMEM((ROWS_PER_W * L,), jnp.int32),
            pltpu.VMEM((LANES * HIST_STRIDE,), jnp.float32),
            pltpu.VMEM((ROWS_PER_W * 6,), jnp.float32),
        ],
        compiler_params=pltpu.CompilerParams(
            use_tc_tiling_on_sc=False, needs_layout_passes=False),
    )(x.reshape(B * L))
    return out.reshape(B, 6)


def kernel(x):
    return _analyze(x)


# speedup vs baseline: 51.4498x; 1.1911x over previous
import functools

import jax
import jax.numpy as jnp
from jax import lax
from jax.experimental import pallas as pl
from jax.experimental.pallas import tpu as pltpu
from jax.experimental.pallas import tpu_sc as plsc

B, L, V = 16384, 200, 40
NC, NS, LANES = 2, 16, 16
NW = NC * NS
ROWS_PER_W = B // NW
GROUPS = ROWS_PER_W // LANES
HIST_STRIDE = 65
POS_UNROLL = 8

_BIG = 1e30


def _body(x_hbm, out_hbm, xv, hist, outv):
    wid = lax.axis_index("s") * NC + lax.axis_index("c")
    row0 = wid * ROWS_PER_W
    pltpu.sync_copy(x_hbm.at[pl.ds(row0 * L, ROWS_PER_W * L)], xv)

    lane = lax.broadcasted_iota(jnp.int32, (LANES,), 0)
    lane_h = lane * HIST_STRIDE
    lane_x = lane * L
    lane_o = lane * 6
    ones = jnp.ones((LANES,), jnp.float32)
    zf = jnp.zeros((LANES,), jnp.float32)

    for k in range(LANES * HIST_STRIDE // LANES):
        hist[pl.ds(k * LANES, LANES)] = zf

    def group_body(g, carry):
        base = lane_x + g * (LANES * L)

        @plsc.parallel_loop(0, L, unroll=POS_UNROLL)
        def _pos(p):
            ch = plsc.load_gather(xv, [base + p])
            plsc.addupdate_scatter(hist, [lane_h + ch], ones)

        lt = zf
        dg = zf
        sp = zf
        uq = zf
        mx = zf
        mn = jnp.full((LANES,), _BIG, jnp.float32)
        for v in range(1, V):
            iv = lane_h + v
            h = plsc.load_gather(hist, [iv])
            plsc.store_scatter(hist, [iv], zf)
            if v < 27:
                lt = lt + h
            elif v < 37:
                dg = dg + h
            else:
                sp = sp + h
            uq = uq + jnp.minimum(h, ones)
            mx = jnp.maximum(mx, h)
            mn = jnp.minimum(mn, jnp.where(h > 0.0, h, _BIG))
        plsc.store_scatter(hist, [lane_h], zf)

        total = lt + dg + sp
        has = total > 0.0
        rec = 1.0 / jnp.where(has, total, 1.0)
        feats = (
            uq * jnp.float32(1.0 / V),
            mx * rec,
            jnp.where(has, mn, 0.0) * rec,
            lt * rec,
            dg * rec,
            sp * rec,
        )
        obase = lane_o + g * (LANES * 6)
        for fi, val in enumerate(feats):
            plsc.store_scatter(outv, [obase + fi], val)
        return carry

    lax.fori_loop(0, GROUPS, group_body, 0)
    pltpu.sync_copy(outv, out_hbm.at[pl.ds(row0 * 6, ROWS_PER_W * 6)])


@jax.jit
def _analyze(x):
    mesh = plsc.VectorSubcoreMesh(core_axis_name="c", subcore_axis_name="s")
    out = pl.kernel(
        _body,
        out_type=jax.ShapeDtypeStruct((B * 6,), jnp.float32),
        mesh=mesh,
        scratch_types=[
            pltpu.VMEM((ROWS_PER_W * L,), jnp.int32),
            pltpu.VMEM((LANES * HIST_STRIDE,), jnp.float32),
            pltpu.VMEM((ROWS_PER_W * 6,), jnp.float32),
        ],
        compiler_params=pltpu.CompilerParams(
            use_tc_tiling_on_sc=False, needs_layout_passes=False),
    )(x.reshape(B * L))
    return out.reshape(B, 6)


def kernel(x):
    return _analyze(x)


# speedup vs baseline: 51.8227x; 1.0072x over previous
import functools

import jax
import jax.numpy as jnp
from jax import lax
from jax.experimental import pallas as pl
from jax.experimental.pallas import tpu as pltpu
from jax.experimental.pallas import tpu_sc as plsc

B, L, V = 16384, 200, 40
NC, NS, LANES = 2, 16, 16
NW = NC * NS
ROWS_PER_W = B // NW
GROUPS = ROWS_PER_W // LANES
HIST_STRIDE = 65
POS_UNROLL = 8

_BIG = 1e30


def _body(x_hbm, out_hbm, xv, hist, outv):
    wid = lax.axis_index("s") * NC + lax.axis_index("c")
    row0 = wid * ROWS_PER_W
    pltpu.sync_copy(x_hbm.at[pl.ds(row0 * L, ROWS_PER_W * L)], xv)

    lane = lax.broadcasted_iota(jnp.int32, (LANES,), 0)
    lane_h = lane * HIST_STRIDE
    lane_x = lane * L
    lane_o = lane * 6
    ones = jnp.ones((LANES,), jnp.float32)
    zf = jnp.zeros((LANES,), jnp.float32)

    for k in range(LANES * HIST_STRIDE // LANES):
        hist[pl.ds(k * LANES, LANES)] = zf

    def group_body(g, carry):
        base = lane_x + g * (LANES * L)

        @plsc.parallel_loop(0, L, unroll=POS_UNROLL)
        def _pos(p):
            q = p + lane
            q = jnp.where(q >= L, q - L, q)
            ch = plsc.load_gather(xv, [base + q])
            plsc.addupdate_scatter(hist, [lane_h + ch], ones)

        lt = zf
        dg = zf
        sp = zf
        uq = zf
        mx = zf
        mn = jnp.full((LANES,), _BIG, jnp.float32)
        for v in range(1, V):
            iv = lane_h + v
            h = plsc.load_gather(hist, [iv])
            plsc.store_scatter(hist, [iv], zf)
            if v < 27:
                lt = lt + h
            elif v < 37:
                dg = dg + h
            else:
                sp = sp + h
            uq = uq + jnp.minimum(h, ones)
            mx = jnp.maximum(mx, h)
            mn = jnp.minimum(mn, jnp.where(h > 0.0, h, _BIG))
        plsc.store_scatter(hist, [lane_h], zf)

        total = lt + dg + sp
        has = total > 0.0
        rec = 1.0 / jnp.where(has, total, 1.0)
        feats = (
            uq * jnp.float32(1.0 / V),
            mx * rec,
            jnp.where(has, mn, 0.0) * rec,
            lt * rec,
            dg * rec,
            sp * rec,
        )
        obase = lane_o + g * (LANES * 6)
        for fi, val in enumerate(feats):
            plsc.store_scatter(outv, [obase + fi], val)
        return carry

    lax.fori_loop(0, GROUPS, group_body, 0)
    pltpu.sync_copy(outv, out_hbm.at[pl.ds(row0 * 6, ROWS_PER_W * 6)])


@jax.jit
def _analyze(x):
    mesh = plsc.VectorSubcoreMesh(core_axis_name="c", subcore_axis_name="s")
    out = pl.kernel(
        _body,
        out_type=jax.ShapeDtypeStruct((B * 6,), jnp.float32),
        mesh=mesh,
        scratch_types=[
            pltpu.VMEM((ROWS_PER_W * L,), jnp.int32),
            pltpu.VMEM((LANES * HIST_STRIDE,), jnp.float32),
            pltpu.VMEM((ROWS_PER_W * 6,), jnp.float32),
        ],
        compiler_params=pltpu.CompilerParams(
            use_tc_tiling_on_sc=False, needs_layout_passes=False),
    )(x.reshape(B * L))
    return out.reshape(B, 6)


def kernel(x):
    return _analyze(x)


# speedup vs baseline: 51.9615x; 1.0027x over previous
import functools

import jax
import jax.numpy as jnp
from jax import lax
from jax.experimental import pallas as pl
from jax.experimental.pallas import tpu as pltpu
from jax.experimental.pallas import tpu_sc as plsc

B, L, V = 16384, 200, 40
NC, NS, LANES = 2, 16, 16
NW = NC * NS
ROWS_PER_W = B // NW
GROUPS = ROWS_PER_W // LANES
HIST_STRIDE = 65
POS_UNROLL = 8

_BIG = 1e30


def _body(x_hbm, out_hbm, xv, hist, outv):
    wid = lax.axis_index("s") * NC + lax.axis_index("c")
    row0 = wid * ROWS_PER_W
    pltpu.sync_copy(x_hbm.at[pl.ds(row0 * L, ROWS_PER_W * L)], xv)

    lane = lax.broadcasted_iota(jnp.int32, (LANES,), 0)
    lane_h = lane * HIST_STRIDE
    lane_x = lane * L
    lane_o = lane * 6
    ones = jnp.ones((LANES,), jnp.float32)
    zf = jnp.zeros((LANES,), jnp.float32)

    for k in range(LANES * HIST_STRIDE // LANES):
        hist[pl.ds(k * LANES, LANES)] = zf

    def group_body(g, carry):
        base = lane_x + g * (LANES * L)

        @plsc.parallel_loop(0, L, unroll=POS_UNROLL)
        def _pos(p):
            q = p + lane
            q = jnp.where(q >= L, q - L, q)
            ch = plsc.load_gather(xv, [base + q])
            plsc.addupdate_scatter(hist, [lane_h + ch], ones)

        lt = zf
        dg = zf
        sp = zf
        uq = zf
        mx = zf
        mn = jnp.full((LANES,), _BIG, jnp.float32)
        for v in range(1, V):
            iv = lane_h + v
            h = plsc.load_gather(hist, [iv])
            plsc.store_scatter(hist, [iv], zf)
            if v < 27:
                lt = lt + h
            elif v < 37:
                dg = dg + h
            else:
                sp = sp + h
            uq = uq + jnp.minimum(h, ones)
            mx = jnp.maximum(mx, h)
            mn = jnp.minimum(mn, jnp.where(h > 0.0, h, _BIG))
        plsc.store_scatter(hist, [lane_h], zf)

        total = lt + dg + sp
        has = total > 0.0
        rec = 1.0 / jnp.where(has, total, 1.0)
        feats = (
            uq * jnp.float32(1.0 / V),
            mx * rec,
            jnp.where(has, mn, 0.0) * rec,
            lt * rec,
            dg * rec,
            sp * rec,
        )
        obase = lane_o + g * (LANES * 6)
        for fi, val in enumerate(feats):
            plsc.store_scatter(outv, [obase + fi], val)
        return carry

    lax.fori_loop(0, GROUPS, group_body, 0)
    pltpu.sync_copy(outv, out_hbm.at[pl.ds(row0 * 6, ROWS_PER_W * 6)])


@jax.jit
def _analyze(x):
    mesh = plsc.VectorSubcoreMesh(core_axis_name="c", subcore_axis_name="s")
    out = pl.kernel(
        _body,
        out_type=jax.ShapeDtypeStruct((B * 6,), jnp.float32),
        mesh=mesh,
        scratch_types=[
            pltpu.VMEM((ROWS_PER_W * L,), jnp.int32),
            pltpu.VMEM((LANES * HIST_STRIDE,), jnp.float32),
            pltpu.VMEM((ROWS_PER_W * 6,), jnp.float32),
        ],
        compiler_params=pltpu.CompilerParams(
            use_tc_tiling_on_sc=True, needs_layout_passes=False),
    )(x.reshape(B * L))
    return out.reshape(B, 6)


def kernel(x):
    return _analyze(x)


# speedup vs baseline: 66.1514x; 1.2731x over previous
import functools

import jax
import jax.numpy as jnp
from jax import lax
from jax.experimental import pallas as pl
from jax.experimental.pallas import tpu as pltpu
from jax.experimental.pallas import tpu_sc as plsc

B, L, V = 16384, 200, 40
NC, NS, LANES = 2, 16, 16
NW = NC * NS
ROWS_PER_W = B // NW
GROUPS = ROWS_PER_W // LANES
HIST_STRIDE = 65
POS_UNROLL = 8

_BIG = 1e30


HALF_ROWS = ROWS_PER_W // 2
HALF_GROUPS = HALF_ROWS // LANES


def _body(x_hbm, out_hbm, xv, hist, outv):
    wid = lax.axis_index("s") * NC + lax.axis_index("c")
    row0 = wid * ROWS_PER_W

    lane = lax.broadcasted_iota(jnp.int32, (LANES,), 0)
    lane_h = lane * HIST_STRIDE
    lane_x = lane * L
    lane_o = lane * 6
    ones = jnp.ones((LANES,), jnp.float32)
    zf = jnp.zeros((LANES,), jnp.float32)

    for k in range(LANES * HIST_STRIDE // LANES):
        hist[pl.ds(k * LANES, LANES)] = zf

    def half_body(h):
        pltpu.sync_copy(x_hbm.at[pl.ds(row0 + h * HALF_ROWS, HALF_ROWS)], xv)
        lax.fori_loop(h * HALF_GROUPS, (h + 1) * HALF_GROUPS, group_body, 0)

    def group_body(g, carry):
        rows = lane + (g % HALF_GROUPS) * LANES

        @plsc.parallel_loop(0, L, unroll=POS_UNROLL)
        def _pos(p):
            q = p + lane
            q = jnp.where(q >= L, q - L, q)
            ch = plsc.load_gather(xv, [rows, q])
            plsc.addupdate_scatter(hist, [lane_h + ch], ones)

        lt = zf
        dg = zf
        sp = zf
        uq = zf
        mx = zf
        mn = jnp.full((LANES,), _BIG, jnp.float32)
        for v in range(1, V):
            iv = lane_h + v
            h = plsc.load_gather(hist, [iv])
            plsc.store_scatter(hist, [iv], zf)
            if v < 27:
                lt = lt + h
            elif v < 37:
                dg = dg + h
            else:
                sp = sp + h
            uq = uq + jnp.minimum(h, ones)
            mx = jnp.maximum(mx, h)
            mn = jnp.minimum(mn, jnp.where(h > 0.0, h, _BIG))
        plsc.store_scatter(hist, [lane_h], zf)

        total = lt + dg + sp
        has = total > 0.0
        rec = 1.0 / jnp.where(has, total, 1.0)
        feats = (
            uq * jnp.float32(1.0 / V),
            mx * rec,
            jnp.where(has, mn, 0.0) * rec,
            lt * rec,
            dg * rec,
            sp * rec,
        )
        obase = lane_o + g * (LANES * 6)
        for fi, val in enumerate(feats):
            plsc.store_scatter(outv, [obase + fi], val)
        return carry

    half_body(0)
    half_body(1)
    pltpu.sync_copy(outv, out_hbm.at[pl.ds(row0 * 6, ROWS_PER_W * 6)])


@jax.jit
def _analyze(x):
    mesh = plsc.VectorSubcoreMesh(core_axis_name="c", subcore_axis_name="s")
    out = pl.kernel(
        _body,
        out_type=jax.ShapeDtypeStruct((B * 6,), jnp.float32),
        mesh=mesh,
        scratch_types=[
            pltpu.VMEM((HALF_ROWS, L), jnp.int32),
            pltpu.VMEM((LANES * HIST_STRIDE,), jnp.float32),
            pltpu.VMEM((ROWS_PER_W * 6,), jnp.float32),
        ],
        compiler_params=pltpu.CompilerParams(
            use_tc_tiling_on_sc=True, needs_layout_passes=False),
    )(x)
    return out.reshape(B, 6)


def kernel(x):
    return _analyze(x)


# speedup vs baseline: 84.0574x; 1.2707x over previous
import functools

import jax
import jax.numpy as jnp
from jax import lax
from jax.experimental import pallas as pl
from jax.experimental.pallas import tpu as pltpu
from jax.experimental.pallas import tpu_sc as plsc

B, L, V = 16384, 200, 40
NC, NS, LANES = 2, 16, 16
NW = NC * NS
ROWS_PER_W = B // NW
GROUPS = ROWS_PER_W // LANES
HIST_STRIDE = 65
POS_UNROLL = 8

_BIG = 1e30


HALF_ROWS = ROWS_PER_W // 2
HALF_GROUPS = HALF_ROWS // LANES


def _body(x_hbm, out_hbm, xv, hist, outv):
    wid = lax.axis_index("s") * NC + lax.axis_index("c")
    row0 = wid * ROWS_PER_W

    lane = lax.broadcasted_iota(jnp.int32, (LANES,), 0)
    lane_h = lane * HIST_STRIDE
    lane_x = lane * L
    lane_o = lane * 6
    ones = jnp.ones((LANES,), jnp.float32)
    zf = jnp.zeros((LANES,), jnp.float32)

    for k in range(LANES * HIST_STRIDE // LANES):
        hist[pl.ds(k * LANES, LANES)] = zf

    def half_body(h):
        pltpu.sync_copy(x_hbm.at[pl.ds(row0 + h * HALF_ROWS, HALF_ROWS)], xv)
        lax.fori_loop(h * HALF_GROUPS, (h + 1) * HALF_GROUPS, group_body, 0)

    def group_body(g, carry):
        rows = lane + (g % HALF_GROUPS) * LANES

        @plsc.parallel_loop(0, L, unroll=POS_UNROLL)
        def _pos(p):
            q = p + lane
            q = jnp.where(q >= L, q - L, q)
            ch = plsc.load_gather(xv, [rows, q])
            plsc.addupdate_scatter(hist, [lane_h + ch], ones)

        lt = zf
        dg = zf
        sp = zf
        uq = zf
        mx = zf
        mn = jnp.full((LANES,), _BIG, jnp.float32)
        for v in range(1, V):
            iv = lane_h + v
            h = plsc.load_gather(hist, [iv])
            plsc.store_scatter(hist, [iv], zf)
            if v < 27:
                lt = lt + h
            elif v < 37:
                dg = dg + h
            else:
                sp = sp + h
            uq = uq + jnp.minimum(h, ones)
            mx = jnp.maximum(mx, h)
            mn = jnp.minimum(mn, jnp.where(h > 0.0, h, _BIG))
        plsc.store_scatter(hist, [lane_h], zf)

        total = lt + dg + sp
        has = total > 0.0
        rec = 1.0 / jnp.where(has, total, 1.0)
        feats = (
            uq * jnp.float32(1.0 / V),
            mx * rec,
            jnp.where(has, mn, 0.0) * rec,
            lt * rec,
            dg * rec,
            sp * rec,
        )
        for fi, val in enumerate(feats):
            outv[fi, pl.ds(g * LANES, LANES)] = val
        return carry

    half_body(0)
    half_body(1)
    pltpu.sync_copy(outv, out_hbm.at[pl.ds(0, 6), pl.ds(row0, ROWS_PER_W)])


@jax.jit
def _analyze(x):
    mesh = plsc.VectorSubcoreMesh(core_axis_name="c", subcore_axis_name="s")
    out = pl.kernel(
        _body,
        out_type=jax.ShapeDtypeStruct((8, B), jnp.float32),
        mesh=mesh,
        scratch_types=[
            pltpu.VMEM((HALF_ROWS, L), jnp.int32),
            pltpu.VMEM((LANES * HIST_STRIDE,), jnp.float32),
            pltpu.VMEM((6, ROWS_PER_W), jnp.float32),
        ],
        compiler_params=pltpu.CompilerParams(
            use_tc_tiling_on_sc=True, needs_layout_passes=False),
    )(x)
    return out[:6].T


def kernel(x):
    return _analyze(x)


# speedup vs baseline: 117.8204x; 1.4017x over previous
import jax
import jax.numpy as jnp
from jax import lax
from jax.experimental import pallas as pl
from jax.experimental.pallas import tpu as pltpu
from jax.experimental.pallas import tpu_sc as plsc

B, L, V = 16384, 200, 40
NC, NS, LANES = 2, 16, 16
NW = NC * NS
ROWS_PER_W = B // NW
GROUPS = ROWS_PER_W // LANES
HIST_STRIDE = 65
POS_UNROLL = 8

_BIG = 1e30


def _body(x_hbm, out_hbm, xv, hist, outv):
    wid = lax.axis_index("s") * NC + lax.axis_index("c")
    row0 = wid * ROWS_PER_W
    pltpu.sync_copy(x_hbm.at[pl.ds(0, L), pl.ds(row0, ROWS_PER_W)], xv)

    lane = lax.broadcasted_iota(jnp.int32, (LANES,), 0)
    lane_h = lane * HIST_STRIDE
    ones = jnp.ones((LANES,), jnp.float32)
    zf = jnp.zeros((LANES,), jnp.float32)

    for k in range(HIST_STRIDE):
        hist[pl.ds(k * LANES, LANES)] = zf

    def group_body(g, carry):
        cols = lane + g * LANES

        @plsc.parallel_loop(0, L, unroll=POS_UNROLL)
        def _pos(p):
            ch = plsc.load_gather(xv, [jnp.full((LANES,), p, jnp.int32), cols])
            plsc.addupdate_scatter(hist, [lane_h + ch], ones)

        lt = zf
        dg = zf
        sp = zf
        uq = zf
        mx = zf
        mn = jnp.full((LANES,), _BIG, jnp.float32)
        for v in range(1, V):
            iv = lane_h + v
            h = plsc.load_gather(hist, [iv])
            plsc.store_scatter(hist, [iv], zf)
            if v < 27:
                lt = lt + h
            elif v < 37:
                dg = dg + h
            else:
                sp = sp + h
            uq = uq + jnp.minimum(h, ones)
            mx = jnp.maximum(mx, h)
            mn = jnp.minimum(mn, jnp.where(h > 0.0, h, _BIG))
        plsc.store_scatter(hist, [lane_h], zf)

        total = lt + dg + sp
        has = total > 0.0
        rec = 1.0 / jnp.where(has, total, 1.0)
        feats = (
            uq * jnp.float32(1.0 / V),
            mx * rec,
            jnp.where(has, mn, 0.0) * rec,
            lt * rec,
            dg * rec,
            sp * rec,
        )
        for fi, val in enumerate(feats):
            outv[fi, pl.ds(g * LANES, LANES)] = val
        return carry

    lax.fori_loop(0, GROUPS, group_body, 0)
    pltpu.sync_copy(outv, out_hbm.at[pl.ds(0, 6), pl.ds(row0, ROWS_PER_W)])


@jax.jit
def _analyze(x):
    mesh = plsc.VectorSubcoreMesh(core_axis_name="c", subcore_axis_name="s")
    out = pl.kernel(
        _body,
        out_type=jax.ShapeDtypeStruct((8, B), jnp.float32),
        mesh=mesh,
        scratch_types=[
            pltpu.VMEM((L, ROWS_PER_W), jnp.int32),
            pltpu.VMEM((LANES * HIST_STRIDE,), jnp.float32),
            pltpu.VMEM((6, ROWS_PER_W), jnp.float32),
        ],
        compiler_params=pltpu.CompilerParams(
            use_tc_tiling_on_sc=True, needs_layout_passes=False),
    )(x.T)
    return out[:6].T


def kernel(x):
    return _analyze(x)
